# baseline (speedup 1.0000x reference)
import functools

import jax
import jax.numpy as jnp
from jax import lax
from jax.experimental import pallas as pl
from jax.experimental.pallas import tpu as pltpu
from jax.experimental.pallas import tpu_sc as plsc


def _build_gather(B, V, D, NC, NS):
    NW = NC * NS
    BPW = B // NW
    CHUNK = 8
    NCH = BPW // CHUNK
    NBUF = 3

    mesh = plsc.VectorSubcoreMesh(core_axis_name="c", subcore_axis_name="s")

    @functools.partial(
        pl.kernel,
        mesh=mesh,
        out_type=jax.ShapeDtypeStruct((B, D), jnp.float32),
        scratch_types=[
            pltpu.VMEM((BPW,), jnp.int32),
            pltpu.VMEM((CHUNK, D), jnp.float32),
            pltpu.VMEM((CHUNK, D), jnp.float32),
            pltpu.VMEM((CHUNK, D), jnp.float32),
            pltpu.SemaphoreType.DMA,
            pltpu.SemaphoreType.DMA,
            pltpu.SemaphoreType.DMA,
            pltpu.SemaphoreType.DMA,
            pltpu.SemaphoreType.DMA,
            pltpu.SemaphoreType.DMA,
        ],
    )
    def k(table_hbm, ids_hbm, out_hbm, idx_v,
          buf0, buf1, buf2, gs0, gs1, gs2, ss0, ss1, ss2):
        wid = lax.axis_index("s") * NC + lax.axis_index("c")
        base = wid * BPW
        pltpu.sync_copy(ids_hbm.at[pl.ds(base, BPW)], idx_v)

        bufs = (buf0, buf1, buf2)
        gsems = (gs0, gs1, gs2)
        ssems = (ss0, ss1, ss2)

        def g_start(j, b):
            pltpu.async_copy(
                table_hbm.at[idx_v.at[pl.ds(j * CHUNK, CHUNK)]], bufs[b], gsems[b]
            )

        def g_wait(j, b):
            pltpu.make_async_copy(
                table_hbm.at[idx_v.at[pl.ds(j * CHUNK, CHUNK)]], bufs[b], gsems[b]
            ).wait()

        def s_start(j, b):
            pltpu.async_copy(bufs[b], out_hbm.at[pl.ds(base + j * CHUNK, CHUNK)], ssems[b])

        def s_wait(j, b):
            pltpu.make_async_copy(
                bufs[b], out_hbm.at[pl.ds(base + j * CHUNK, CHUNK)], ssems[b]
            ).wait()

        g_start(0, 0)
        g_start(1, 1)
        g_wait(0, 0); s_start(0, 0); g_start(2, 2)
        g_wait(1, 1); s_start(1, 1); s_wait(0, 0); g_start(3, 0)

        def body(j, b):
            g_wait(j, b)
            s_start(j, b)
            s_wait(j - 1, (b + 2) % NBUF)
            g_start(j + 2, (b + 2) % NBUF)

        F = (NCH - 4) // NBUF

        def group(gi, carry):
            j0 = 2 + gi * NBUF
            for u in range(NBUF):
                body(j0 + u, (2 + u) % NBUF)
            return carry

        lax.fori_loop(0, F, group, 0)

        for j in range(2 + NBUF * F, NCH - 2):
            body(j, j % NBUF)

        for j in range(NCH - 2, NCH):
            b = j % NBUF
            g_wait(j, b)
            s_start(j, b)
            s_wait(j - 1, (j - 1) % NBUF)
        s_wait(NCH - 1, (NCH - 1) % NBUF)

    return k


def kernel(input_ids, embed_weight):
    V, D = embed_weight.shape
    B = input_ids.size
    info = plsc.get_sparse_core_info()
    ids_flat = input_ids.reshape(-1).astype(jnp.int32)
    gather = _build_gather(B, V, D, info.num_cores, info.num_subcores)
    out = gather(embed_weight, ids_flat)
    return out.reshape(*input_ids.shape, D)
